# baseline (device time: 55820 ns/iter reference)
import jax
import jax.numpy as jnp
from jax import lax
from jax.experimental import pallas as pl
from jax.experimental.pallas import tpu as pltpu

M_HALF = 1024
D = 1024
EPS = 1e-6


def kernel(partial, gamma):
    partial2d = partial.reshape(2 * M_HALF, D)
    gamma2d = gamma.reshape(1, D)

    def body(p_ref, g_ref, out_ref, comm_ref, send_sem, recv_sem):
        my_x = lax.axis_index("x")
        my_y = lax.axis_index("y")
        my_z = lax.axis_index("z")
        nbr = (my_x, 1 - my_y, my_z)

        barrier_sem = pltpu.get_barrier_semaphore()
        pl.semaphore_signal(
            barrier_sem, inc=1, device_id=nbr,
            device_id_type=pl.DeviceIdType.MESH,
        )
        pl.semaphore_wait(barrier_sem, 1)

        nbr_row0 = (1 - my_y) * M_HALF
        rdma = pltpu.make_async_remote_copy(
            src_ref=p_ref.at[pl.ds(nbr_row0, M_HALF), :],
            dst_ref=comm_ref,
            send_sem=send_sem,
            recv_sem=recv_sem,
            device_id=nbr,
            device_id_type=pl.DeviceIdType.MESH,
        )
        rdma.start()
        rdma.wait()

        my_row0 = my_y * M_HALF
        y_sum = p_ref[pl.ds(my_row0, M_HALF), :] + comm_ref[:, :]
        ms = jnp.mean(y_sum * y_sum, axis=-1, keepdims=True)
        out_ref[:, :] = y_sum * lax.rsqrt(ms + EPS) * g_ref[:, :]

    return pl.pallas_call(
        body,
        out_shape=jax.ShapeDtypeStruct((M_HALF, D), jnp.float32),
        in_specs=[
            pl.BlockSpec(memory_space=pltpu.VMEM),
            pl.BlockSpec(memory_space=pltpu.VMEM),
        ],
        out_specs=pl.BlockSpec(memory_space=pltpu.VMEM),
        scratch_shapes=[
            pltpu.VMEM((M_HALF, D), jnp.float32),
            pltpu.SemaphoreType.DMA,
            pltpu.SemaphoreType.DMA,
        ],
        compiler_params=pltpu.CompilerParams(collective_id=0),
    )(partial2d, gamma2d)


# device time: 34728 ns/iter; 1.6073x vs baseline; 1.6073x over previous
import jax
import jax.numpy as jnp
from jax import lax
from jax.experimental import pallas as pl
from jax.experimental.pallas import tpu as pltpu

M_HALF = 1024
D = 1024
Q = 256
K = 4
CH = Q // K
H = Q // 2
EPS = 1e-6
_MESH = pl.DeviceIdType.MESH


def kernel(partial, gamma):
    partial2d = partial.reshape(2 * M_HALF, D)
    gamma2d = gamma.reshape(1, D)

    def body(p_ref, g_ref, out_ref, comm_ref,
             s1_send, s1_recv,
             dn_send, dp_send, d_recv_prev, d_recv_next,
             f_send, f_recv_prev, f_recv_next):
        my_x = lax.axis_index("x")
        my_y = lax.axis_index("y")
        my_z = lax.axis_index("z")
        nbr_y = (my_x, 1 - my_y, my_z)

        rho = 2 * my_z + (my_x + my_z) % 2
        rho_next = (rho + 1) % 4
        rho_prev = (rho + 3) % 4
        rho_opp = (rho + 2) % 4
        is_even = rho % 2 == 0
        nxt_x = jnp.where(is_even, 1 - my_x, my_x)
        nxt_z = jnp.where(is_even, my_z, 1 - my_z)
        prv_x = jnp.where(is_even, my_x, 1 - my_x)
        prv_z = jnp.where(is_even, 1 - my_z, my_z)
        nbr_next = (nxt_x, my_y, nxt_z)
        nbr_prev = (prv_x, my_y, prv_z)

        barrier_sem = pltpu.get_barrier_semaphore()
        for nbr in (nbr_y, nbr_next, nbr_prev):
            pl.semaphore_signal(barrier_sem, inc=1, device_id=nbr,
                                device_id_type=_MESH)
        pl.semaphore_wait(barrier_sem, 3)

        nbr_base = (1 - my_y) * M_HALF + Q * rho
        p1 = []
        for k in range(K):
            rdma = pltpu.make_async_remote_copy(
                src_ref=p_ref.at[pl.ds(nbr_base + k * CH, CH), :],
                dst_ref=comm_ref.at[k],
                send_sem=s1_send.at[k], recv_sem=s1_recv.at[k],
                device_id=nbr_y, device_id_type=_MESH,
            )
            rdma.start()
            p1.append(rdma)

        my_base = my_y * M_HALF + Q * rho
        p2 = []
        for k in range(K):
            p1[k].wait_recv()
            y_sum = p_ref[pl.ds(my_base + k * CH, CH), :] + comm_ref[k]
            ms = jnp.mean(y_sum * y_sum, axis=-1, keepdims=True)
            out_ref[pl.ds(Q * rho + k * CH, CH), :] = (
                y_sum * lax.rsqrt(ms + EPS) * g_ref[:, :]
            )
            for dst, ssem, rsem in (
                (nbr_next, dn_send, d_recv_prev),
                (nbr_prev, dp_send, d_recv_next),
            ):
                rdma = pltpu.make_async_remote_copy(
                    src_ref=out_ref.at[pl.ds(Q * rho + k * CH, CH), :],
                    dst_ref=out_ref.at[pl.ds(Q * rho + k * CH, CH), :],
                    send_sem=ssem.at[k], recv_sem=rsem.at[k],
                    device_id=dst, device_id_type=_MESH,
                )
                rdma.start()
                p2.append(rdma)

        def inbound(dst_off, nrows, rsem, src_dev):
            return pltpu.make_async_remote_copy(
                src_ref=out_ref.at[pl.ds(0, nrows), :],
                dst_ref=out_ref.at[pl.ds(dst_off, nrows), :],
                send_sem=s1_send.at[0], recv_sem=rsem,
                device_id=src_dev, device_id_type=_MESH,
            )

        for k in range(K // 2):
            inbound(Q * rho_prev + k * CH, CH, d_recv_prev.at[k],
                    nbr_prev).wait_recv()
        fwd_n = pltpu.make_async_remote_copy(
            src_ref=out_ref.at[pl.ds(Q * rho_prev, H), :],
            dst_ref=out_ref.at[pl.ds(Q * rho_prev, H), :],
            send_sem=f_send.at[0], recv_sem=f_recv_prev,
            device_id=nbr_next, device_id_type=_MESH,
        )
        fwd_n.start()
        for k in range(K // 2, K):
            inbound(Q * rho_next + k * CH, CH, d_recv_next.at[k],
                    nbr_next).wait_recv()
        fwd_p = pltpu.make_async_remote_copy(
            src_ref=out_ref.at[pl.ds(Q * rho_next + H, H), :],
            dst_ref=out_ref.at[pl.ds(Q * rho_next + H, H), :],
            send_sem=f_send.at[1], recv_sem=f_recv_next,
            device_id=nbr_prev, device_id_type=_MESH,
        )
        fwd_p.start()

        for k in range(K // 2, K):
            inbound(Q * rho_prev + k * CH, CH, d_recv_prev.at[k],
                    nbr_prev).wait_recv()
        for k in range(K // 2):
            inbound(Q * rho_next + k * CH, CH, d_recv_next.at[k],
                    nbr_next).wait_recv()
        inbound(Q * rho_opp, H, f_recv_prev, nbr_prev).wait_recv()
        inbound(Q * rho_opp + H, H, f_recv_next, nbr_next).wait_recv()

        for r in p1:
            r.wait_send()
        for r in p2:
            r.wait_send()
        fwd_n.wait_send()
        fwd_p.wait_send()

    return pl.pallas_call(
        body,
        out_shape=jax.ShapeDtypeStruct((M_HALF, D), jnp.float32),
        in_specs=[
            pl.BlockSpec(memory_space=pltpu.VMEM),
            pl.BlockSpec(memory_space=pltpu.VMEM),
        ],
        out_specs=pl.BlockSpec(memory_space=pltpu.VMEM),
        scratch_shapes=[
            pltpu.VMEM((K, CH, D), jnp.float32),
            pltpu.SemaphoreType.DMA((K,)),
            pltpu.SemaphoreType.DMA((K,)),
            pltpu.SemaphoreType.DMA((K,)),
            pltpu.SemaphoreType.DMA((K,)),
            pltpu.SemaphoreType.DMA((K,)),
            pltpu.SemaphoreType.DMA((K,)),
            pltpu.SemaphoreType.DMA((2,)),
            pltpu.SemaphoreType.DMA,
            pltpu.SemaphoreType.DMA,
        ],
        compiler_params=pltpu.CompilerParams(collective_id=0),
    )(partial2d, gamma2d)


# device time: 24977 ns/iter; 2.2349x vs baseline; 1.3904x over previous
import jax
import jax.numpy as jnp
from jax import lax
from jax.experimental import pallas as pl
from jax.experimental.pallas import tpu as pltpu

M_HALF = 1024
D = 1024
Q = 256
K = 4
CH = Q // K
H = Q // 2
EPS = 1e-6
_MESH = pl.DeviceIdType.MESH


def kernel(partial, gamma):
    partial2d = partial.reshape(2 * M_HALF, D)
    gamma2d = gamma.reshape(1, D)

    def body(p_ref, g_ref, out_ref, stage_ref, comm_ref, pc_ref,
             s1_send, s1_recv,
             dn_send, dp_send, d_recv_prev, d_recv_next,
             f_send, f_recv_prev, f_recv_next):
        my_x = lax.axis_index("x")
        my_y = lax.axis_index("y")
        my_z = lax.axis_index("z")
        nbr_y = (my_x, 1 - my_y, my_z)

        rho = 2 * my_z + (my_x + my_z) % 2
        rho_next = (rho + 1) % 4
        rho_prev = (rho + 3) % 4
        rho_opp = (rho + 2) % 4
        is_even = rho % 2 == 0
        nxt_x = jnp.where(is_even, 1 - my_x, my_x)
        nxt_z = jnp.where(is_even, my_z, 1 - my_z)
        prv_x = jnp.where(is_even, my_x, 1 - my_x)
        prv_z = jnp.where(is_even, 1 - my_z, my_z)
        nbr_next = (nxt_x, my_y, nxt_z)
        nbr_prev = (prv_x, my_y, prv_z)

        barrier_sem = pltpu.get_barrier_semaphore()
        for nbr in (nbr_y, nbr_next, nbr_prev):
            pl.semaphore_signal(barrier_sem, inc=1, device_id=nbr,
                                device_id_type=_MESH)
        pl.semaphore_wait(barrier_sem, 3)

        nbr_base = (1 - my_y) * M_HALF + Q * rho
        p1 = []
        for k in range(K):
            stage_ref[k] = p_ref[
                pl.ds(nbr_base + k * CH, CH), :
            ].astype(jnp.bfloat16)
            rdma = pltpu.make_async_remote_copy(
                src_ref=stage_ref.at[k],
                dst_ref=comm_ref.at[k],
                send_sem=s1_send.at[k], recv_sem=s1_recv.at[k],
                device_id=nbr_y, device_id_type=_MESH,
            )
            rdma.start()
            p1.append(rdma)

        my_base = my_y * M_HALF + Q * rho
        p2 = []
        for k in range(K):
            p1[k].wait_recv()
            y_sum = (p_ref[pl.ds(my_base + k * CH, CH), :]
                     + comm_ref[k].astype(jnp.float32))
            ms = jnp.mean(y_sum * y_sum, axis=-1, keepdims=True)
            chunk = y_sum * lax.rsqrt(ms + EPS) * g_ref[:, :]
            out_ref[pl.ds(Q * rho + k * CH, CH), :] = chunk
            pc_ref[pl.ds(Q * rho + k * CH, CH), :] = chunk.astype(
                jnp.bfloat16)
            for dst, ssem, rsem in (
                (nbr_next, dn_send, d_recv_prev),
                (nbr_prev, dp_send, d_recv_next),
            ):
                rdma = pltpu.make_async_remote_copy(
                    src_ref=pc_ref.at[pl.ds(Q * rho + k * CH, CH), :],
                    dst_ref=pc_ref.at[pl.ds(Q * rho + k * CH, CH), :],
                    send_sem=ssem.at[k], recv_sem=rsem.at[k],
                    device_id=dst, device_id_type=_MESH,
                )
                rdma.start()
                p2.append(rdma)

        def inbound(dst_off, nrows, rsem, src_dev):
            return pltpu.make_async_remote_copy(
                src_ref=pc_ref.at[pl.ds(0, nrows), :],
                dst_ref=pc_ref.at[pl.ds(dst_off, nrows), :],
                send_sem=s1_send.at[0], recv_sem=rsem,
                device_id=src_dev, device_id_type=_MESH,
            )

        for k in range(K // 2):
            inbound(Q * rho_prev + k * CH, CH, d_recv_prev.at[k],
                    nbr_prev).wait_recv()
        fwd_n = pltpu.make_async_remote_copy(
            src_ref=pc_ref.at[pl.ds(Q * rho_prev, H), :],
            dst_ref=pc_ref.at[pl.ds(Q * rho_prev, H), :],
            send_sem=f_send.at[0], recv_sem=f_recv_prev,
            device_id=nbr_next, device_id_type=_MESH,
        )
        fwd_n.start()
        for k in range(K // 2, K):
            inbound(Q * rho_next + k * CH, CH, d_recv_next.at[k],
                    nbr_next).wait_recv()
        fwd_p = pltpu.make_async_remote_copy(
            src_ref=pc_ref.at[pl.ds(Q * rho_next + H, H), :],
            dst_ref=pc_ref.at[pl.ds(Q * rho_next + H, H), :],
            send_sem=f_send.at[1], recv_sem=f_recv_next,
            device_id=nbr_prev, device_id_type=_MESH,
        )
        fwd_p.start()

        for k in range(K // 2, K):
            inbound(Q * rho_prev + k * CH, CH, d_recv_prev.at[k],
                    nbr_prev).wait_recv()
        out_ref[pl.ds(Q * rho_prev, Q), :] = pc_ref[
            pl.ds(Q * rho_prev, Q), :
        ].astype(jnp.float32)
        for k in range(K // 2):
            inbound(Q * rho_next + k * CH, CH, d_recv_next.at[k],
                    nbr_next).wait_recv()
        out_ref[pl.ds(Q * rho_next, Q), :] = pc_ref[
            pl.ds(Q * rho_next, Q), :
        ].astype(jnp.float32)
        inbound(Q * rho_opp, H, f_recv_prev, nbr_prev).wait_recv()
        inbound(Q * rho_opp + H, H, f_recv_next, nbr_next).wait_recv()
        out_ref[pl.ds(Q * rho_opp, Q), :] = pc_ref[
            pl.ds(Q * rho_opp, Q), :
        ].astype(jnp.float32)

        for r in p1:
            r.wait_send()
        for r in p2:
            r.wait_send()
        fwd_n.wait_send()
        fwd_p.wait_send()

    return pl.pallas_call(
        body,
        out_shape=jax.ShapeDtypeStruct((M_HALF, D), jnp.float32),
        in_specs=[
            pl.BlockSpec(memory_space=pltpu.VMEM),
            pl.BlockSpec(memory_space=pltpu.VMEM),
        ],
        out_specs=pl.BlockSpec(memory_space=pltpu.VMEM),
        scratch_shapes=[
            pltpu.VMEM((K, CH, D), jnp.bfloat16),
            pltpu.VMEM((K, CH, D), jnp.bfloat16),
            pltpu.VMEM((M_HALF, D), jnp.bfloat16),
            pltpu.SemaphoreType.DMA((K,)),
            pltpu.SemaphoreType.DMA((K,)),
            pltpu.SemaphoreType.DMA((K,)),
            pltpu.SemaphoreType.DMA((K,)),
            pltpu.SemaphoreType.DMA((K,)),
            pltpu.SemaphoreType.DMA((K,)),
            pltpu.SemaphoreType.DMA((2,)),
            pltpu.SemaphoreType.DMA,
            pltpu.SemaphoreType.DMA,
        ],
        compiler_params=pltpu.CompilerParams(collective_id=0),
    )(partial2d, gamma2d)


# device time: 24147 ns/iter; 2.3117x vs baseline; 1.0344x over previous
import jax
import jax.numpy as jnp
from jax import lax
from jax.experimental import pallas as pl
from jax.experimental.pallas import tpu as pltpu

M_HALF = 1024
D = 1024
Q = 256
K = 8
CH = Q // K
H = Q // 2
EPS = 1e-6
_MESH = pl.DeviceIdType.MESH


def kernel(partial, gamma):
    partial2d = partial.reshape(2 * M_HALF, D)
    gamma2d = gamma.reshape(1, D)

    def body(p_ref, g_ref, out_ref, stage_ref, comm_ref, pc_ref,
             s1_send, s1_recv,
             dn_send, dp_send, d_recv_prev, d_recv_next,
             f_send, f_recv_prev, f_recv_next):
        my_x = lax.axis_index("x")
        my_y = lax.axis_index("y")
        my_z = lax.axis_index("z")
        nbr_y = (my_x, 1 - my_y, my_z)

        rho = 2 * my_z + (my_x + my_z) % 2
        rho_next = (rho + 1) % 4
        rho_prev = (rho + 3) % 4
        rho_opp = (rho + 2) % 4
        is_even = rho % 2 == 0
        nxt_x = jnp.where(is_even, 1 - my_x, my_x)
        nxt_z = jnp.where(is_even, my_z, 1 - my_z)
        prv_x = jnp.where(is_even, my_x, 1 - my_x)
        prv_z = jnp.where(is_even, 1 - my_z, my_z)
        nbr_next = (nxt_x, my_y, nxt_z)
        nbr_prev = (prv_x, my_y, prv_z)

        barrier_sem = pltpu.get_barrier_semaphore()
        for nbr in (nbr_y, nbr_next, nbr_prev):
            pl.semaphore_signal(barrier_sem, inc=1, device_id=nbr,
                                device_id_type=_MESH)
        pl.semaphore_wait(barrier_sem, 3)

        nbr_base = (1 - my_y) * M_HALF + Q * rho
        p1 = []
        for k in range(K):
            stage_ref[k] = p_ref[
                pl.ds(nbr_base + k * CH, CH), :
            ].astype(jnp.bfloat16)
            rdma = pltpu.make_async_remote_copy(
                src_ref=stage_ref.at[k],
                dst_ref=comm_ref.at[k],
                send_sem=s1_send.at[k], recv_sem=s1_recv.at[k],
                device_id=nbr_y, device_id_type=_MESH,
            )
            rdma.start()
            p1.append(rdma)

        my_base = my_y * M_HALF + Q * rho
        p2 = []
        for k in range(K):
            p1[k].wait_recv()
            y_sum = (p_ref[pl.ds(my_base + k * CH, CH), :]
                     + comm_ref[k].astype(jnp.float32))
            ms = jnp.mean(y_sum * y_sum, axis=-1, keepdims=True)
            chunk = y_sum * lax.rsqrt(ms + EPS) * g_ref[:, :]
            out_ref[pl.ds(Q * rho + k * CH, CH), :] = chunk
            pc_ref[pl.ds(Q * rho + k * CH, CH), :] = chunk.astype(
                jnp.bfloat16)
            for dst, ssem, rsem in (
                (nbr_next, dn_send, d_recv_prev),
                (nbr_prev, dp_send, d_recv_next),
            ):
                rdma = pltpu.make_async_remote_copy(
                    src_ref=pc_ref.at[pl.ds(Q * rho + k * CH, CH), :],
                    dst_ref=pc_ref.at[pl.ds(Q * rho + k * CH, CH), :],
                    send_sem=ssem.at[k], recv_sem=rsem.at[k],
                    device_id=dst, device_id_type=_MESH,
                )
                rdma.start()
                p2.append(rdma)

        def inbound(dst_off, nrows, rsem, src_dev):
            return pltpu.make_async_remote_copy(
                src_ref=pc_ref.at[pl.ds(0, nrows), :],
                dst_ref=pc_ref.at[pl.ds(dst_off, nrows), :],
                send_sem=s1_send.at[0], recv_sem=rsem,
                device_id=src_dev, device_id_type=_MESH,
            )

        for k in range(K // 2):
            inbound(Q * rho_prev + k * CH, CH, d_recv_prev.at[k],
                    nbr_prev).wait_recv()
        fwd_n = pltpu.make_async_remote_copy(
            src_ref=pc_ref.at[pl.ds(Q * rho_prev, H), :],
            dst_ref=pc_ref.at[pl.ds(Q * rho_prev, H), :],
            send_sem=f_send.at[0], recv_sem=f_recv_prev,
            device_id=nbr_next, device_id_type=_MESH,
        )
        fwd_n.start()
        for k in range(K // 2, K):
            inbound(Q * rho_next + k * CH, CH, d_recv_next.at[k],
                    nbr_next).wait_recv()
        fwd_p = pltpu.make_async_remote_copy(
            src_ref=pc_ref.at[pl.ds(Q * rho_next + H, H), :],
            dst_ref=pc_ref.at[pl.ds(Q * rho_next + H, H), :],
            send_sem=f_send.at[1], recv_sem=f_recv_next,
            device_id=nbr_prev, device_id_type=_MESH,
        )
        fwd_p.start()

        for k in range(K // 2, K):
            inbound(Q * rho_prev + k * CH, CH, d_recv_prev.at[k],
                    nbr_prev).wait_recv()
        out_ref[pl.ds(Q * rho_prev, Q), :] = pc_ref[
            pl.ds(Q * rho_prev, Q), :
        ].astype(jnp.float32)
        for k in range(K // 2):
            inbound(Q * rho_next + k * CH, CH, d_recv_next.at[k],
                    nbr_next).wait_recv()
        out_ref[pl.ds(Q * rho_next, Q), :] = pc_ref[
            pl.ds(Q * rho_next, Q), :
        ].astype(jnp.float32)
        inbound(Q * rho_opp, H, f_recv_prev, nbr_prev).wait_recv()
        out_ref[pl.ds(Q * rho_opp, H), :] = pc_ref[
            pl.ds(Q * rho_opp, H), :
        ].astype(jnp.float32)
        inbound(Q * rho_opp + H, H, f_recv_next, nbr_next).wait_recv()
        out_ref[pl.ds(Q * rho_opp + H, H), :] = pc_ref[
            pl.ds(Q * rho_opp + H, H), :
        ].astype(jnp.float32)

        for r in p1:
            r.wait_send()
        for r in p2:
            r.wait_send()
        fwd_n.wait_send()
        fwd_p.wait_send()

    return pl.pallas_call(
        body,
        out_shape=jax.ShapeDtypeStruct((M_HALF, D), jnp.float32),
        in_specs=[
            pl.BlockSpec(memory_space=pltpu.VMEM),
            pl.BlockSpec(memory_space=pltpu.VMEM),
        ],
        out_specs=pl.BlockSpec(memory_space=pltpu.VMEM),
        scratch_shapes=[
            pltpu.VMEM((K, CH, D), jnp.bfloat16),
            pltpu.VMEM((K, CH, D), jnp.bfloat16),
            pltpu.VMEM((M_HALF, D), jnp.bfloat16),
            pltpu.SemaphoreType.DMA((K,)),
            pltpu.SemaphoreType.DMA((K,)),
            pltpu.SemaphoreType.DMA((K,)),
            pltpu.SemaphoreType.DMA((K,)),
            pltpu.SemaphoreType.DMA((K,)),
            pltpu.SemaphoreType.DMA((K,)),
            pltpu.SemaphoreType.DMA((2,)),
            pltpu.SemaphoreType.DMA,
            pltpu.SemaphoreType.DMA,
        ],
        compiler_params=pltpu.CompilerParams(collective_id=0),
    )(partial2d, gamma2d)
